# Initial kernel scaffold; baseline (speedup 1.0000x reference)
import jax
import jax.numpy as jnp
from jax.experimental import pallas as pl


def kernel(node_states, edge_index, edge_states, params):
    raise NotImplementedError("write your pallas kernel here")


# speedup vs baseline: 24.4727x; 24.4727x over previous
import functools
import math

import jax
import jax.numpy as jnp
from jax import lax
from jax.experimental import pallas as pl
from jax.experimental.pallas import tpu as pltpu
from jax.experimental.pallas import tpu_sc as plsc

N = 10000
E = 320000
D = 128
H = 8
DH = 16

NC = 2
NS = 16
NW = NC * NS
EPW = E // NW
CH = 80
NCHUNK = EPW // CH
NZC = 400
NZN = N // NZC

_mesh = plsc.VectorSubcoreMesh(
    core_axis_name="c", subcore_axis_name="s", num_cores=NC, num_subcores=NS)
_sc_params = pltpu.CompilerParams(use_tc_tiling_on_sc=False)


def _ln_block(x, g, b):
  m = jnp.mean(x, axis=-1, keepdims=True)
  v = jnp.mean((x - m) ** 2, axis=-1, keepdims=True)
  return (x - m) * jax.lax.rsqrt(v + 1e-5) * g + b


def _node_pre_body(x_ref, g_ref, b_ref, wq_ref, bq_ref, wk_ref, bk_ref,
                   wv_ref, bv_ref, q_ref, k_ref, vm_ref):
  x = x_ref[...]
  xn = _ln_block(x, g_ref[...], b_ref[...])
  q = jnp.dot(xn, wq_ref[...], preferred_element_type=jnp.float32) + bq_ref[...]
  k = jnp.dot(xn, wk_ref[...], preferred_element_type=jnp.float32) + bk_ref[...]
  v = jnp.dot(xn, wv_ref[...], preferred_element_type=jnp.float32) + bv_ref[...]
  q_ref[...] = q
  k_ref[...] = k
  vm_ref[...] = v * xn


def _node_pre(x, g, b, wq, bq, wk, bk, wv, bv):
  blk = 1000
  grid = N // blk
  row = pl.BlockSpec((blk, D), lambda i: (i, 0))
  full = pl.BlockSpec((D, D), lambda i: (0, 0))
  vec = pl.BlockSpec((D,), lambda i: (0,))
  return pl.pallas_call(
      _node_pre_body,
      grid=(grid,),
      in_specs=[row, vec, vec, full, vec, full, vec, full, vec],
      out_specs=[row, row, row],
      out_shape=[jax.ShapeDtypeStruct((N, D), jnp.float32)] * 3,
  )(x, g, b, wq, bq, wk, bk, wv, bv)


def _edge_num_body(qs_ref, kd_ref, e_ref, g_ref, b_ref, wes_ref, bes_ref,
                   num_ref):
  blk = qs_ref.shape[0]
  en = _ln_block(e_ref[...], g_ref[...], b_ref[...])
  esh = (jnp.dot(en, wes_ref[...], preferred_element_type=jnp.float32)
         + bes_ref[...])
  qk = (qs_ref[...] * kd_ref[...]).reshape(blk, H, DH).sum(-1)
  ep = (esh * en).reshape(blk, H, DH).sum(-1)
  s = (qk + ep) * (1.0 / math.sqrt(DH))
  num = jnp.exp(s)
  num_ref[...] = jnp.concatenate(
      [num, jnp.zeros((blk, 16 - H), jnp.float32)], axis=1)


def _edge_num(qs, kd, e, g, b, wes, bes):
  blk = 2000
  grid = E // blk
  row = pl.BlockSpec((blk, D), lambda i: (i, 0))
  full = pl.BlockSpec((D, D), lambda i: (0, 0))
  vec = pl.BlockSpec((D,), lambda i: (0,))
  out = pl.BlockSpec((blk, 16), lambda i: (i, 0))
  return pl.pallas_call(
      _edge_num_body,
      grid=(grid,),
      in_specs=[row, row, row, vec, vec, full, vec],
      out_specs=out,
      out_shape=jax.ShapeDtypeStruct((E, 16), jnp.float32),
  )(qs, kd, e, g, b, wes, bes)


def _gather3_body(q_hbm, k_hbm, vm_hbm, src_hbm, dst_hbm,
                  qs_out, kd_out, vmd_out,
                  src_v, dst_v, bq, bk, bv, sem):
  wid = lax.axis_index("s") * NC + lax.axis_index("c")
  base0 = wid * EPW

  def body(i, _):
    base = base0 + i * CH
    pltpu.sync_copy(src_hbm.at[pl.ds(base, CH)], src_v)
    pltpu.sync_copy(dst_hbm.at[pl.ds(base, CH)], dst_v)
    cq = pltpu.async_copy(q_hbm.at[src_v], bq, sem)
    ck = pltpu.async_copy(k_hbm.at[dst_v], bk, sem)
    cv = pltpu.async_copy(vm_hbm.at[dst_v], bv, sem)
    cq.wait()
    ck.wait()
    cv.wait()
    pltpu.sync_copy(bq, qs_out.at[pl.ds(base, CH)])
    pltpu.sync_copy(bk, kd_out.at[pl.ds(base, CH)])
    pltpu.sync_copy(bv, vmd_out.at[pl.ds(base, CH)])
    return 0

  lax.fori_loop(0, NCHUNK, body, 0)


def _gather3(q, k, vm, src, dst):
  f = pl.kernel(
      _gather3_body,
      out_type=[jax.ShapeDtypeStruct((E, D), jnp.float32)] * 3,
      mesh=_mesh,
      compiler_params=_sc_params,
      scratch_types=[
          pltpu.VMEM((CH,), jnp.int32),
          pltpu.VMEM((CH,), jnp.int32),
          pltpu.VMEM((CH, D), jnp.float32),
          pltpu.VMEM((CH, D), jnp.float32),
          pltpu.VMEM((CH, D), jnp.float32),
          pltpu.SemaphoreType.DMA,
      ],
  )
  return f(q, k, vm, src, dst)


def _segsum_body(vals_hbm, idx_hbm, out_hbm, idx_v, val_v, zb, acc):
  cid = lax.axis_index("c")
  sid = lax.axis_index("s")
  wid = sid * NC + cid
  w = acc.shape[1]

  zb[...] = jnp.zeros(zb.shape, jnp.float32)
  for j in range((NZN + NS - 1) // NS):
    ci = sid + j * NS
    @pl.when(ci < NZN)
    def _():
      pltpu.sync_copy(zb, acc.at[pl.ds(ci * NZC, NZC)])
  plsc.subcore_barrier()

  base0 = wid * EPW

  def body(i, _):
    base = base0 + i * CH
    pltpu.sync_copy(idx_hbm.at[pl.ds(base, CH)], idx_v)
    pltpu.sync_copy(vals_hbm.at[pl.ds(base, CH)], val_v)
    pltpu.sync_copy(val_v, acc.at[idx_v], add=True)
    return 0

  lax.fori_loop(0, NCHUNK, body, 0)
  plsc.subcore_barrier()
  for j in range((NZN + NS - 1) // NS):
    ci = sid + j * NS
    @pl.when(ci < NZN)
    def _():
      pltpu.sync_copy(acc.at[pl.ds(ci * NZC, NZC)],
                      out_hbm.at[cid].at[pl.ds(ci * NZC, NZC)])


def _segsum(vals, idx, width):
  f = pl.kernel(
      functools.partial(_segsum_body),
      out_type=jax.ShapeDtypeStruct((NC, N, width), jnp.float32),
      mesh=_mesh,
      compiler_params=_sc_params,
      scratch_types=[
          pltpu.VMEM((CH,), jnp.int32),
          pltpu.VMEM((CH, width), jnp.float32),
          pltpu.VMEM((NZC, width), jnp.float32),
          pltpu.VMEM_SHARED((N, width), jnp.float32),
      ],
  )
  return f(vals, idx)


EPT = E // NS
NCH2 = EPT // CH
W2 = D // NC


def _segsum_split_body(vals_hbm, idx_hbm, out_hbm, idx_v, val_v, zb, acc):
  cid = lax.axis_index("c")
  sid = lax.axis_index("s")
  c0 = cid * W2

  zb[...] = jnp.zeros(zb.shape, jnp.float32)
  for j in range((NZN + NS - 1) // NS):
    ci = sid + j * NS
    @pl.when(ci < NZN)
    def _():
      pltpu.sync_copy(zb, acc.at[pl.ds(ci * NZC, NZC)])
  plsc.subcore_barrier()

  base0 = sid * EPT

  def body(i, _):
    base = base0 + i * CH
    pltpu.sync_copy(idx_hbm.at[pl.ds(base, CH)], idx_v)
    pltpu.sync_copy(vals_hbm.at[pl.ds(base, CH), pl.ds(c0, W2)], val_v)
    pltpu.sync_copy(val_v, acc.at[idx_v], add=True)
    return 0

  lax.fori_loop(0, NCH2, body, 0)
  plsc.subcore_barrier()
  for j in range((NZN + NS - 1) // NS):
    ci = sid + j * NS
    @pl.when(ci < NZN)
    def _():
      pltpu.sync_copy(acc.at[pl.ds(ci * NZC, NZC)],
                      out_hbm.at[pl.ds(ci * NZC, NZC), pl.ds(c0, W2)])


def _segsum_split(vals, idx):
  f = pl.kernel(
      _segsum_split_body,
      out_type=jax.ShapeDtypeStruct((N, D), jnp.float32),
      mesh=_mesh,
      compiler_params=_sc_params,
      scratch_types=[
          pltpu.VMEM((CH,), jnp.int32),
          pltpu.VMEM((CH, W2), jnp.float32),
          pltpu.VMEM((NZC, W2), jnp.float32),
          pltpu.VMEM_SHARED((N, W2), jnp.float32),
      ],
  )
  return f(vals, idx)


def _dgather_body(den_hbm, src_hbm, out_hbm, src_v, buf, sem):
  wid = lax.axis_index("s") * NC + lax.axis_index("c")
  base0 = wid * EPW

  def body(i, _):
    base = base0 + i * CH
    pltpu.sync_copy(src_hbm.at[pl.ds(base, CH)], src_v)
    pltpu.async_copy(den_hbm.at[src_v], buf, sem).wait()
    pltpu.sync_copy(buf, out_hbm.at[pl.ds(base, CH)])
    return 0

  lax.fori_loop(0, NCHUNK, body, 0)


def _dgather(den, src):
  f = pl.kernel(
      _dgather_body,
      out_type=jax.ShapeDtypeStruct((E, 16), jnp.float32),
      mesh=_mesh,
      compiler_params=_sc_params,
      scratch_types=[
          pltpu.VMEM((CH,), jnp.int32),
          pltpu.VMEM((CH, 16), jnp.float32),
          pltpu.SemaphoreType.DMA,
      ],
  )
  return f(den, src)


def _psum_body(p_ref, o_ref):
  o_ref[...] = p_ref[0] + p_ref[1]


def _partial_sum(p, width):
  blk = 1000
  grid = N // blk
  return pl.pallas_call(
      _psum_body,
      grid=(grid,),
      in_specs=[pl.BlockSpec((NC, blk, width), lambda i: (0, i, 0))],
      out_specs=pl.BlockSpec((blk, width), lambda i: (i, 0)),
      out_shape=jax.ShapeDtypeStruct((N, width), jnp.float32),
  )(p)


def _edge_post_body(num_ref, den_ref, vmd_ref, e_ref, weo_ref, beo_ref,
                    g_ref, b_ref, w1_ref, b1_ref, w2_ref, b2_ref,
                    wt_ref, oe_ref, attn_ref):
  blk = num_ref.shape[0]
  num = num_ref[...][:, :H]
  den = den_ref[...][:, :H]
  attn = num / (den + 1e-12)
  attn_ref[...] = attn
  aw = jnp.broadcast_to(attn[:, :, None], (blk, H, DH)).reshape(blk, D)
  wt_ref[...] = aw * vmd_ref[...]
  eau = (jnp.dot(attn, weo_ref[...], preferred_element_type=jnp.float32)
         + beo_ref[...])
  es = e_ref[...] + eau
  x = _ln_block(es, g_ref[...], b_ref[...])
  h1 = jax.nn.relu(
      jnp.dot(x, w1_ref[...], preferred_element_type=jnp.float32) + b1_ref[...])
  ef = (jnp.dot(h1, w2_ref[...], preferred_element_type=jnp.float32)
        + b2_ref[...])
  oe_ref[...] = es + ef


def _edge_post(num, den_s, vmd, e, weo, beo, g, b, w1, b1, w2, b2):
  blk = 1000
  grid = E // blk
  row = pl.BlockSpec((blk, D), lambda i: (i, 0))
  n16 = pl.BlockSpec((blk, 16), lambda i: (i, 0))
  vec = pl.BlockSpec((D,), lambda i: (0,))
  return pl.pallas_call(
      _edge_post_body,
      grid=(grid,),
      in_specs=[n16, n16, row, row,
                pl.BlockSpec((H, D), lambda i: (0, 0)), vec,
                vec, vec,
                pl.BlockSpec((D, 2 * D), lambda i: (0, 0)),
                pl.BlockSpec((2 * D,), lambda i: (0,)),
                pl.BlockSpec((2 * D, D), lambda i: (0, 0)), vec],
      out_specs=[row, row, pl.BlockSpec((blk, H), lambda i: (i, 0))],
      out_shape=[jax.ShapeDtypeStruct((E, D), jnp.float32),
                 jax.ShapeDtypeStruct((E, D), jnp.float32),
                 jax.ShapeDtypeStruct((E, H), jnp.float32)],
  )(num, den_s, vmd, e, weo, beo, g, b, w1, b1, w2, b2)


def _node_post_body(p_ref, x_ref, wno_ref, bno_ref, g_ref, b_ref,
                    w1_ref, b1_ref, w2_ref, b2_ref, o_ref):
  agg = p_ref[...]
  nau = (jnp.dot(agg, wno_ref[...], preferred_element_type=jnp.float32)
         + bno_ref[...])
  ns = x_ref[...] + nau
  x = _ln_block(ns, g_ref[...], b_ref[...])
  h1 = jax.nn.relu(
      jnp.dot(x, w1_ref[...], preferred_element_type=jnp.float32) + b1_ref[...])
  nf = (jnp.dot(h1, w2_ref[...], preferred_element_type=jnp.float32)
        + b2_ref[...])
  o_ref[...] = ns + nf


def _node_post(p, x, wno, bno, g, b, w1, b1, w2, b2):
  blk = 1000
  grid = N // blk
  row = pl.BlockSpec((blk, D), lambda i: (i, 0))
  full = pl.BlockSpec((D, D), lambda i: (0, 0))
  vec = pl.BlockSpec((D,), lambda i: (0,))
  return pl.pallas_call(
      _node_post_body,
      grid=(grid,),
      in_specs=[row,
                row, full, vec, vec, vec,
                pl.BlockSpec((D, 2 * D), lambda i: (0, 0)),
                pl.BlockSpec((2 * D,), lambda i: (0,)),
                pl.BlockSpec((2 * D, D), lambda i: (0, 0)), vec],
      out_specs=row,
      out_shape=jax.ShapeDtypeStruct((N, D), jnp.float32),
  )(p, x, wno, bno, g, b, w1, b1, w2, b2)


@jax.jit
def kernel(node_states, edge_index, edge_states, params):
  p = params
  src = edge_index[0]
  dst = edge_index[1]

  q, k, vm = _node_pre(node_states, p['nln1_g'], p['nln1_b'],
                       p['wq'], p['bq'], p['wk'], p['bk'], p['wv'], p['bv'])

  qs, kd, vmd = _gather3(q, k, vm, src, dst)

  num = _edge_num(qs, kd, edge_states, p['eln1_g'], p['eln1_b'],
                  p['wes'], p['bes'])

  den_p = _segsum(num, src, 16)
  den = _partial_sum(den_p, 16)
  den_s = _dgather(den, src)

  weighted, out_edges, attn = _edge_post(
      num, den_s, vmd, edge_states, p['weo'], p['beo'],
      p['eln2_g'], p['eln2_b'], p['ef1_w'], p['ef1_b'], p['ef2_w'], p['ef2_b'])

  agg = _segsum_split(weighted, src)

  out_nodes = _node_post(agg, node_states, p['wno'], p['bno'],
                         p['nln2_g'], p['nln2_b'],
                         p['nf1_w'], p['nf1_b'], p['nf2_w'], p['nf2_b'])

  return (out_nodes, out_edges, attn)


# speedup vs baseline: 44.6834x; 1.8258x over previous
import functools
import math

import jax
import jax.numpy as jnp
from jax import lax
from jax.experimental import pallas as pl
from jax.experimental.pallas import tpu as pltpu
from jax.experimental.pallas import tpu_sc as plsc

N = 10000
E = 320000
D = 128
H = 8
DH = 16

NC = 2
NS = 16
NW = NC * NS
EPW = E // NW
CH = 80
NCHUNK = EPW // CH
NZC = 400
NZN = N // NZC

_mesh = plsc.VectorSubcoreMesh(
    core_axis_name="c", subcore_axis_name="s", num_cores=NC, num_subcores=NS)
_sc_params = pltpu.CompilerParams(use_tc_tiling_on_sc=False)


def _ln_block(x, g, b):
  ones = jnp.ones((D, 1), jnp.float32)
  s1 = jnp.dot(x, ones, preferred_element_type=jnp.float32)
  s2 = jnp.dot(x * x, ones, preferred_element_type=jnp.float32)
  m = s1 * (1.0 / D)
  var = s2 * (1.0 / D) - m * m
  return (x - m) * jax.lax.rsqrt(var + 1e-5) * g + b


def _head_sel():
  r = lax.broadcasted_iota(jnp.int32, (D, 16), 0) // DH
  c = lax.broadcasted_iota(jnp.int32, (D, 16), 1)
  return (r == c).astype(jnp.float32)


def _node_pre_body(x_ref, g_ref, b_ref, wq_ref, bq_ref, wk_ref, bk_ref,
                   wv_ref, bv_ref, q_ref, k_ref, vm_ref):
  x = x_ref[...]
  xn = _ln_block(x, g_ref[...], b_ref[...])
  q = jnp.dot(xn, wq_ref[...], preferred_element_type=jnp.float32) + bq_ref[...]
  k = jnp.dot(xn, wk_ref[...], preferred_element_type=jnp.float32) + bk_ref[...]
  v = jnp.dot(xn, wv_ref[...], preferred_element_type=jnp.float32) + bv_ref[...]
  q_ref[...] = q
  k_ref[...] = k
  vm_ref[...] = v * xn


def _node_pre(x, g, b, wq, bq, wk, bk, wv, bv):
  blk = 1000
  grid = N // blk
  row = pl.BlockSpec((blk, D), lambda i: (i, 0))
  full = pl.BlockSpec((D, D), lambda i: (0, 0))
  vec = pl.BlockSpec((D,), lambda i: (0,))
  return pl.pallas_call(
      _node_pre_body,
      grid=(grid,),
      in_specs=[row, vec, vec, full, vec, full, vec, full, vec],
      out_specs=[row, row, row],
      out_shape=[jax.ShapeDtypeStruct((N, D), jnp.float32)] * 3,
  )(x, g, b, wq, bq, wk, bk, wv, bv)


def _edge_ep_body(e_ref, g_ref, b_ref, wes_ref, bes_ref, ep_ref):
  en = _ln_block(e_ref[...], g_ref[...], b_ref[...])
  esh = (jnp.dot(en, wes_ref[...], preferred_element_type=jnp.float32)
         + bes_ref[...])
  ep_ref[...] = jnp.dot(esh * en, _head_sel(),
                        preferred_element_type=jnp.float32)


def _edge_ep(e, g, b, wes, bes):
  blk = 2000
  grid = E // blk
  row = pl.BlockSpec((blk, D), lambda i: (i, 0))
  full = pl.BlockSpec((D, D), lambda i: (0, 0))
  vec = pl.BlockSpec((D,), lambda i: (0,))
  out = pl.BlockSpec((blk, 16), lambda i: (i, 0))
  return pl.pallas_call(
      _edge_ep_body,
      grid=(grid,),
      in_specs=[row, vec, vec, full, vec],
      out_specs=out,
      out_shape=jax.ShapeDtypeStruct((E, 16), jnp.float32),
  )(e, g, b, wes, bes)


def _edge_num_body(qs_ref, kd_ref, ep_ref, num_ref):
  qk = jnp.dot(qs_ref[...] * kd_ref[...], _head_sel(),
               preferred_element_type=jnp.float32)
  num_ref[...] = jnp.exp((qk + ep_ref[...]) * (1.0 / math.sqrt(DH)))


def _edge_num(qs, kd, ep):
  blk = 2000
  grid = E // blk
  row = pl.BlockSpec((blk, D), lambda i: (i, 0))
  n16 = pl.BlockSpec((blk, 16), lambda i: (i, 0))
  return pl.pallas_call(
      _edge_num_body,
      grid=(grid,),
      in_specs=[row, row, n16],
      out_specs=n16,
      out_shape=jax.ShapeDtypeStruct((E, 16), jnp.float32),
  )(qs, kd, ep)


def _gather3_body(q_hbm, k_hbm, vm_hbm, src_hbm, dst_hbm,
                  qs_out, kd_out, vmd_out,
                  src_v, dst_v, bq, bk, bv, sem):
  wid = lax.axis_index("s") * NC + lax.axis_index("c")
  base0 = wid * EPW

  def body(i, _):
    base = base0 + i * CH
    pltpu.sync_copy(src_hbm.at[pl.ds(base, CH)], src_v)
    pltpu.sync_copy(dst_hbm.at[pl.ds(base, CH)], dst_v)
    cq = pltpu.async_copy(q_hbm.at[src_v], bq, sem)
    ck = pltpu.async_copy(k_hbm.at[dst_v], bk, sem)
    cv = pltpu.async_copy(vm_hbm.at[dst_v], bv, sem)
    cq.wait()
    ck.wait()
    cv.wait()
    pltpu.sync_copy(bq, qs_out.at[pl.ds(base, CH)])
    pltpu.sync_copy(bk, kd_out.at[pl.ds(base, CH)])
    pltpu.sync_copy(bv, vmd_out.at[pl.ds(base, CH)])
    return 0

  lax.fori_loop(0, NCHUNK, body, 0)


def _gather3(q, k, vm, src, dst):
  f = pl.kernel(
      _gather3_body,
      out_type=[jax.ShapeDtypeStruct((E, D), jnp.float32)] * 3,
      mesh=_mesh,
      compiler_params=_sc_params,
      scratch_types=[
          pltpu.VMEM((CH,), jnp.int32),
          pltpu.VMEM((CH,), jnp.int32),
          pltpu.VMEM((CH, D), jnp.float32),
          pltpu.VMEM((CH, D), jnp.float32),
          pltpu.VMEM((CH, D), jnp.float32),
          pltpu.SemaphoreType.DMA,
      ],
  )
  return f(q, k, vm, src, dst)


def _segsum_body(vals_hbm, idx_hbm, out_hbm, idx_v, val_v, zb, acc):
  cid = lax.axis_index("c")
  sid = lax.axis_index("s")
  wid = sid * NC + cid
  w = acc.shape[1]

  zb[...] = jnp.zeros(zb.shape, jnp.float32)
  for j in range((NZN + NS - 1) // NS):
    ci = sid + j * NS
    @pl.when(ci < NZN)
    def _():
      pltpu.sync_copy(zb, acc.at[pl.ds(ci * NZC, NZC)])
  plsc.subcore_barrier()

  base0 = wid * EPW

  def body(i, _):
    base = base0 + i * CH
    pltpu.sync_copy(idx_hbm.at[pl.ds(base, CH)], idx_v)
    pltpu.sync_copy(vals_hbm.at[pl.ds(base, CH)], val_v)
    pltpu.sync_copy(val_v, acc.at[idx_v], add=True)
    return 0

  lax.fori_loop(0, NCHUNK, body, 0)
  plsc.subcore_barrier()
  for j in range((NZN + NS - 1) // NS):
    ci = sid + j * NS
    @pl.when(ci < NZN)
    def _():
      pltpu.sync_copy(acc.at[pl.ds(ci * NZC, NZC)],
                      out_hbm.at[cid].at[pl.ds(ci * NZC, NZC)])


def _segsum(vals, idx, width):
  f = pl.kernel(
      functools.partial(_segsum_body),
      out_type=jax.ShapeDtypeStruct((NC, N, width), jnp.float32),
      mesh=_mesh,
      compiler_params=_sc_params,
      scratch_types=[
          pltpu.VMEM((CH,), jnp.int32),
          pltpu.VMEM((CH, width), jnp.float32),
          pltpu.VMEM((NZC, width), jnp.float32),
          pltpu.VMEM_SHARED((N, width), jnp.float32),
      ],
  )
  return f(vals, idx)


EPT = E // NS
NCH2 = EPT // CH
W2 = D // NC


def _segsum_split_body(vals_hbm, idx_hbm, out_hbm, idx_v, val_v, zb, acc):
  cid = lax.axis_index("c")
  sid = lax.axis_index("s")
  c0 = cid * W2

  zb[...] = jnp.zeros(zb.shape, jnp.float32)
  for j in range((NZN + NS - 1) // NS):
    ci = sid + j * NS
    @pl.when(ci < NZN)
    def _():
      pltpu.sync_copy(zb, acc.at[pl.ds(ci * NZC, NZC)])
  plsc.subcore_barrier()

  base0 = sid * EPT

  def body(i, _):
    base = base0 + i * CH
    pltpu.sync_copy(idx_hbm.at[pl.ds(base, CH)], idx_v)
    pltpu.sync_copy(vals_hbm.at[pl.ds(base, CH), pl.ds(c0, W2)], val_v)
    pltpu.sync_copy(val_v, acc.at[idx_v], add=True)
    return 0

  lax.fori_loop(0, NCH2, body, 0)
  plsc.subcore_barrier()
  for j in range((NZN + NS - 1) // NS):
    ci = sid + j * NS
    @pl.when(ci < NZN)
    def _():
      pltpu.sync_copy(acc.at[pl.ds(ci * NZC, NZC)],
                      out_hbm.at[pl.ds(ci * NZC, NZC), pl.ds(c0, W2)])


def _segsum_split(vals, idx):
  f = pl.kernel(
      _segsum_split_body,
      out_type=jax.ShapeDtypeStruct((N, D), jnp.float32),
      mesh=_mesh,
      compiler_params=_sc_params,
      scratch_types=[
          pltpu.VMEM((CH,), jnp.int32),
          pltpu.VMEM((CH, W2), jnp.float32),
          pltpu.VMEM((NZC, W2), jnp.float32),
          pltpu.VMEM_SHARED((N, W2), jnp.float32),
      ],
  )
  return f(vals, idx)


def _dgather_body(den_hbm, src_hbm, out_hbm, src_v, buf, sem):
  wid = lax.axis_index("s") * NC + lax.axis_index("c")
  base0 = wid * EPW

  def body(i, _):
    base = base0 + i * CH
    pltpu.sync_copy(src_hbm.at[pl.ds(base, CH)], src_v)
    pltpu.async_copy(den_hbm.at[src_v], buf, sem).wait()
    pltpu.sync_copy(buf, out_hbm.at[pl.ds(base, CH)])
    return 0

  lax.fori_loop(0, NCHUNK, body, 0)


def _dgather(den, src):
  f = pl.kernel(
      _dgather_body,
      out_type=jax.ShapeDtypeStruct((E, 16), jnp.float32),
      mesh=_mesh,
      compiler_params=_sc_params,
      scratch_types=[
          pltpu.VMEM((CH,), jnp.int32),
          pltpu.VMEM((CH, 16), jnp.float32),
          pltpu.SemaphoreType.DMA,
      ],
  )
  return f(den, src)


def _psum_body(p_ref, o_ref):
  o_ref[...] = p_ref[0] + p_ref[1]


def _partial_sum(p, width):
  blk = 1000
  grid = N // blk
  return pl.pallas_call(
      _psum_body,
      grid=(grid,),
      in_specs=[pl.BlockSpec((NC, blk, width), lambda i: (0, i, 0))],
      out_specs=pl.BlockSpec((blk, width), lambda i: (i, 0)),
      out_shape=jax.ShapeDtypeStruct((N, width), jnp.float32),
  )(p)


def _edge_attn_body(num_ref, den_ref, vmd_ref, wt_ref, attn_ref):
  attn16 = num_ref[...] / (den_ref[...] + 1e-12)
  attn_ref[...] = attn16[:, :H]
  aw = jnp.dot(attn16, _head_sel().T, preferred_element_type=jnp.float32)
  wt_ref[...] = aw * vmd_ref[...]


def _edge_attn(num, den_s, vmd):
  blk = 2000
  grid = E // blk
  row = pl.BlockSpec((blk, D), lambda i: (i, 0))
  n16 = pl.BlockSpec((blk, 16), lambda i: (i, 0))
  return pl.pallas_call(
      _edge_attn_body,
      grid=(grid,),
      in_specs=[n16, n16, row],
      out_specs=[row, pl.BlockSpec((blk, H), lambda i: (i, 0))],
      out_shape=[jax.ShapeDtypeStruct((E, D), jnp.float32),
                 jax.ShapeDtypeStruct((E, H), jnp.float32)],
  )(num, den_s, vmd)


def _edge_ffn_body(attn_ref, e_ref, weo_ref, beo_ref,
                   g_ref, b_ref, w1_ref, b1_ref, w2_ref, b2_ref, oe_ref):
  eau = (jnp.dot(attn_ref[...], weo_ref[...],
                 preferred_element_type=jnp.float32) + beo_ref[...])
  es = e_ref[...] + eau
  x = _ln_block(es, g_ref[...], b_ref[...])
  h1 = jax.nn.relu(
      jnp.dot(x, w1_ref[...], preferred_element_type=jnp.float32) + b1_ref[...])
  ef = (jnp.dot(h1, w2_ref[...], preferred_element_type=jnp.float32)
        + b2_ref[...])
  oe_ref[...] = es + ef


def _edge_ffn(attn, e, weo, beo, g, b, w1, b1, w2, b2):
  blk = 2000
  grid = E // blk
  row = pl.BlockSpec((blk, D), lambda i: (i, 0))
  vec = pl.BlockSpec((D,), lambda i: (0,))
  return pl.pallas_call(
      _edge_ffn_body,
      grid=(grid,),
      in_specs=[pl.BlockSpec((blk, H), lambda i: (i, 0)), row,
                pl.BlockSpec((H, D), lambda i: (0, 0)), vec,
                vec, vec,
                pl.BlockSpec((D, 2 * D), lambda i: (0, 0)),
                pl.BlockSpec((2 * D,), lambda i: (0,)),
                pl.BlockSpec((2 * D, D), lambda i: (0, 0)), vec],
      out_specs=row,
      out_shape=jax.ShapeDtypeStruct((E, D), jnp.float32),
  )(attn, e, weo, beo, g, b, w1, b1, w2, b2)


def _node_post_body(p_ref, x_ref, wno_ref, bno_ref, g_ref, b_ref,
                    w1_ref, b1_ref, w2_ref, b2_ref, o_ref):
  agg = p_ref[...]
  nau = (jnp.dot(agg, wno_ref[...], preferred_element_type=jnp.float32)
         + bno_ref[...])
  ns = x_ref[...] + nau
  x = _ln_block(ns, g_ref[...], b_ref[...])
  h1 = jax.nn.relu(
      jnp.dot(x, w1_ref[...], preferred_element_type=jnp.float32) + b1_ref[...])
  nf = (jnp.dot(h1, w2_ref[...], preferred_element_type=jnp.float32)
        + b2_ref[...])
  o_ref[...] = ns + nf


def _node_post(p, x, wno, bno, g, b, w1, b1, w2, b2):
  blk = 1000
  grid = N // blk
  row = pl.BlockSpec((blk, D), lambda i: (i, 0))
  full = pl.BlockSpec((D, D), lambda i: (0, 0))
  vec = pl.BlockSpec((D,), lambda i: (0,))
  return pl.pallas_call(
      _node_post_body,
      grid=(grid,),
      in_specs=[row,
                row, full, vec, vec, vec,
                pl.BlockSpec((D, 2 * D), lambda i: (0, 0)),
                pl.BlockSpec((2 * D,), lambda i: (0,)),
                pl.BlockSpec((2 * D, D), lambda i: (0, 0)), vec],
      out_specs=row,
      out_shape=jax.ShapeDtypeStruct((N, D), jnp.float32),
  )(p, x, wno, bno, g, b, w1, b1, w2, b2)


@jax.jit
def kernel(node_states, edge_index, edge_states, params):
  p = params
  src = edge_index[0]
  dst = edge_index[1]

  q, k, vm = _node_pre(node_states, p['nln1_g'], p['nln1_b'],
                       p['wq'], p['bq'], p['wk'], p['bk'], p['wv'], p['bv'])

  ep = _edge_ep(edge_states, p['eln1_g'], p['eln1_b'], p['wes'], p['bes'])

  qs, kd, vmd = _gather3(q, k, vm, src, dst)

  num = _edge_num(qs, kd, ep)

  den_p = _segsum(num, src, 16)
  den = _partial_sum(den_p, 16)
  den_s = _dgather(den, src)

  weighted, attn = _edge_attn(num, den_s, vmd)

  out_edges = _edge_ffn(attn, edge_states, p['weo'], p['beo'],
                        p['eln2_g'], p['eln2_b'],
                        p['ef1_w'], p['ef1_b'], p['ef2_w'], p['ef2_b'])

  agg = _segsum_split(weighted, src)

  out_nodes = _node_post(agg, node_states, p['wno'], p['bno'],
                         p['nln2_g'], p['nln2_b'],
                         p['nf1_w'], p['nf1_b'], p['nf2_w'], p['nf2_b'])

  return (out_nodes, out_edges, attn)
